# baseline (speedup 1.0000x reference)
import functools

import jax
import jax.numpy as jnp
from jax import lax
from jax.experimental import pallas as pl
from jax.experimental.pallas import tpu as pltpu, tpu_sc as plsc

D_MODEL = 1024
BATCH = 4
SEQ = 4096
NC, NS, NLANE = 2, 16, 16
NW = NC * NS
SPW = SEQ // NW
R = 16
STEPS = SPW // R
TILES = STEPS * BATCH
TW = R * D_MODEL
VECS = TW // NLANE
NBUF = 4


@functools.partial(
    pl.kernel,
    out_type=jax.ShapeDtypeStruct((BATCH * SEQ * D_MODEL,), jnp.float32),
    mesh=plsc.VectorSubcoreMesh(core_axis_name="c", subcore_axis_name="s"),
    scratch_types=(
        [pltpu.VMEM((TW,), jnp.float32) for _ in range(NBUF)]
        + [pltpu.VMEM((TW,), jnp.float32)]
        + [pltpu.SemaphoreType.DMA for _ in range(2 * NBUF)]
    ),
)
def _sc_add(x_hbm, pos_hbm, out_hbm, *refs):
    xb = refs[:NBUF]
    pbuf = refs[NBUF]
    sl = refs[NBUF + 1 : NBUF + 1 + NBUF]
    ss = refs[NBUF + 1 + NBUF :]
    wid = lax.axis_index("s") * NC + lax.axis_index("c")
    s_base = wid * SPW

    def x_off(t):
        step, b = t // BATCH, t % BATCH
        return (b * SEQ + s_base + step * R) * D_MODEL

    def accumulate(buf):
        def body(i, _):
            off = i * (16 * NLANE)
            for u in range(16):
                o = off + u * NLANE
                plsc.addupdate(buf.at[pl.ds(o, NLANE)], pbuf[pl.ds(o, NLANE)])
            return 0

        lax.fori_loop(0, VECS // 16, body, 0)

    load_h = [None] * NBUF
    store_h = [None] * NBUF
    for p in range(NBUF - 1):
        load_h[p] = pltpu.async_copy(x_hbm.at[pl.ds(x_off(p), TW)], xb[p], sl[p])
    for t in range(TILES):
        slot = t % NBUF
        if t + NBUF - 1 < TILES:
            nxt = (t + NBUF - 1) % NBUF
            if store_h[nxt] is not None:
                store_h[nxt].wait()
            load_h[nxt] = pltpu.async_copy(
                x_hbm.at[pl.ds(x_off(t + NBUF - 1), TW)], xb[nxt], sl[nxt]
            )
        if t % BATCH == 0:
            p0 = (s_base + (t // BATCH) * R) * D_MODEL
            pltpu.sync_copy(pos_hbm.at[pl.ds(p0, TW)], pbuf)
        load_h[slot].wait()
        accumulate(xb[slot])
        store_h[slot] = pltpu.async_copy(
            xb[slot], out_hbm.at[pl.ds(x_off(t), TW)], ss[slot]
        )
    for p in range(NBUF):
        if store_h[p] is not None:
            store_h[p].wait()


def kernel(x, pos_table):
    batch, seq_len, d_model = x.shape
    out2 = _sc_add(x.reshape(-1), pos_table.reshape(-1))
    return out2.reshape(batch, seq_len, d_model)


# speedup vs baseline: 4.9193x; 4.9193x over previous
import jax
import jax.numpy as jnp
from jax.experimental import pallas as pl


_BS = 512


def _add_kernel(x_ref, pos_ref, out_ref):
    out_ref[...] = x_ref[...] + pos_ref[...]


def kernel(x, pos_table):
    batch, seq_len, d_model = x.shape
    bs = _BS
    num_s = seq_len // bs

    out = pl.pallas_call(
        _add_kernel,
        grid=(num_s,),
        in_specs=[
            pl.BlockSpec((batch, bs, d_model), lambda i: (0, i, 0)),
            pl.BlockSpec((bs, d_model), lambda i: (i, 0)),
        ],
        out_specs=pl.BlockSpec((batch, bs, d_model), lambda i: (0, i, 0)),
        out_shape=jax.ShapeDtypeStruct(x.shape, x.dtype),
    )(x, pos_table)
    return out
